# baseline (speedup 1.0000x reference)
import functools

import jax
import jax.numpy as jnp
from jax import lax
from jax.experimental import pallas as pl
from jax.experimental.pallas import tpu as pltpu
from jax.experimental.pallas import tpu_sc as plsc

B, N, C, NS = 4, 2048, 128, 16
C2 = C // 2
P = B * N
M = NS * P
EPS = 1e-5

_INTERP = False


def _pack_bf16(g):
    u = lax.bitcast_convert_type(g, jnp.uint32)
    r = (u + 0x7FFF + ((u >> 16) & 1)) >> 16
    lo = r[:, :C2]
    hi = r[:, C2:]
    return lax.bitcast_convert_type(lo | (hi << 16), jnp.float32)


def _unpack_bf16(gp):
    u = lax.bitcast_convert_type(gp, jnp.uint32)
    lo = lax.bitcast_convert_type(u << 16, jnp.float32)
    hi = lax.bitcast_convert_type(u & jnp.uint32(0xFFFF0000), jnp.float32)
    return jnp.concatenate([lo, hi], axis=-1)


def _proj_body(pos1_ref, pos2_ref, f1_ref, f2_ref, wp_ref, wf1_ref, wf2_ref,
               g_ref, h_ref):
    dn = (((0,), (1,)), ((), ()))
    g = lax.dot_general(f2_ref[0], wf2_ref[...], dn,
                        preferred_element_type=jnp.float32)
    g += lax.dot_general(pos2_ref[0], wp_ref[...], dn,
                         preferred_element_type=jnp.float32)
    g_ref[...] = g
    h = lax.dot_general(f1_ref[0], wf1_ref[...], dn,
                        preferred_element_type=jnp.float32)
    h -= lax.dot_general(pos1_ref[0], wp_ref[...], dn,
                         preferred_element_type=jnp.float32)
    h_ref[...] = h


def _proj(pos1, pos2, f1, f2, wp, wf1, wf2):
    return pl.pallas_call(
        _proj_body,
        grid=(B,),
        in_specs=[
            pl.BlockSpec((1, 3, N), lambda b: (b, 0, 0)),
            pl.BlockSpec((1, 3, N), lambda b: (b, 0, 0)),
            pl.BlockSpec((1, C, N), lambda b: (b, 0, 0)),
            pl.BlockSpec((1, C, N), lambda b: (b, 0, 0)),
            pl.BlockSpec((C, 3), lambda b: (0, 0)),
            pl.BlockSpec((C, C), lambda b: (0, 0)),
            pl.BlockSpec((C, C), lambda b: (0, 0)),
        ],
        out_specs=[
            pl.BlockSpec((N, C), lambda b: (b, 0)),
            pl.BlockSpec((N, C), lambda b: (b, 0)),
        ],
        out_shape=[
            jax.ShapeDtypeStruct((P, C), jnp.float32),
            jax.ShapeDtypeStruct((P, C), jnp.float32),
        ],
        interpret=_INTERP,
    )(pos1, pos2, f1, f2, wp, wf1, wf2)


_RB = 512


def _knn_body(p1_ref, p2_ref, out_ref):
    b = pl.program_id(0)
    p1 = p1_ref[0]
    p2 = p2_ref[0]
    d = -2.0 * lax.dot_general(p1, p2, (((0,), (0,)), ((), ())),
                               preferred_element_type=jnp.float32)
    d += jnp.sum(p2 * p2, axis=0)[None, :]
    d = d.T
    iota = lax.broadcasted_iota(jnp.int32, (N, _RB), 0)
    inf = jnp.float32(jnp.inf)
    for s in range(NS):
        am = jnp.argmin(d, axis=0).astype(jnp.int32)
        out_ref[s, :] = am + b * N
        d = jnp.where(iota == am[None, :], inf, d)


def _knn(pos1, pos2):
    return pl.pallas_call(
        _knn_body,
        grid=(B, N // _RB),
        in_specs=[
            pl.BlockSpec((1, 3, _RB), lambda b, i: (b, 0, i)),
            pl.BlockSpec((1, 3, N), lambda b, i: (b, 0, 0)),
        ],
        out_specs=pl.BlockSpec((NS, _RB), lambda b, i: (0, b * (N // _RB) + i)),
        out_shape=jax.ShapeDtypeStruct((NS, P), jnp.int32),
        interpret=_INTERP,
    )(pos1, pos2)


_NC_SC, _NSUB_SC = 2, 16
_NW = _NC_SC * _NSUB_SC
_ROWS_W = M // _NW
_CHUNK = 128
_NCHUNK = _ROWS_W // _CHUNK


def _gather_sc(table, idx2d):
    mesh = plsc.VectorSubcoreMesh(core_axis_name="c", subcore_axis_name="s")

    @functools.partial(
        pl.kernel, mesh=mesh,
        out_type=jax.ShapeDtypeStruct((M, C), jnp.float32),
        scratch_types=[
            pltpu.VMEM((_NCHUNK, _CHUNK), jnp.int32),
            pltpu.VMEM((_CHUNK, C), jnp.float32),
            pltpu.VMEM((_CHUNK, C), jnp.float32),
            pltpu.SemaphoreType.DMA,
            pltpu.SemaphoreType.DMA,
        ],
    )
    def k(table_hbm, idx_hbm, out_hbm, idx_v, buf0, buf1, sem0, sem1):
        wid = lax.axis_index("s") * _NC_SC + lax.axis_index("c")
        pltpu.sync_copy(idx_hbm.at[pl.ds(wid * _NCHUNK, _NCHUNK)], idx_v)
        out_base = wid * _ROWS_W

        def body(j2, _):
            j0 = j2 * 2
            cp0 = pltpu.async_copy(table_hbm.at[idx_v.at[j0]], buf0, sem0)
            cp1 = pltpu.async_copy(table_hbm.at[idx_v.at[j0 + 1]], buf1, sem1)
            cp0.wait()
            pltpu.sync_copy(buf0, out_hbm.at[pl.ds(out_base + j0 * _CHUNK,
                                                   _CHUNK)])
            cp1.wait()
            pltpu.sync_copy(buf1, out_hbm.at[pl.ds(out_base + (j0 + 1) * _CHUNK,
                                                   _CHUNK)])
            return 0

        lax.fori_loop(0, _NCHUNK // 2, body, 0)

    return k(table, idx2d)


def _affine(g, b, s, q):
    mean = s / jnp.float32(M)
    var = q / jnp.float32(M) - mean * mean
    a = g * lax.rsqrt(var + EPS)
    c = b - mean * a
    return a, c


_PB = 1024

_STATS_OUT_SPECS = [
    pl.BlockSpec((1, C), lambda i, s: (0, 0)),
    pl.BlockSpec((1, C), lambda i, s: (0, 0)),
]
_STATS_OUT_SHAPE = [
    jax.ShapeDtypeStruct((1, C), jnp.float32),
    jax.ShapeDtypeStruct((1, C), jnp.float32),
]
_GG_SPEC = pl.BlockSpec((1, _PB, C), lambda i, s: (s, i, 0))
_GGP_SPEC = pl.BlockSpec((1, _PB, C2), lambda i, s: (s, i, 0))
_HT_SPEC = pl.BlockSpec((_PB, C), lambda i, s: (i, 0))
_W_SPEC = pl.BlockSpec((C, C), lambda i, s: (0, 0))
_V_SPEC = pl.BlockSpec((1, C), lambda i, s: (0, 0))


def _acc_stats(sum_ref, sq_ref, y):
    @pl.when((pl.program_id(0) == 0) & (pl.program_id(1) == 0))
    def _():
        sum_ref[...] = jnp.zeros_like(sum_ref)
        sq_ref[...] = jnp.zeros_like(sq_ref)

    sum_ref[0, :] += jnp.sum(y, axis=0)
    sq_ref[0, :] += jnp.sum(y * y, axis=0)


def _bn_relu_mm(y, w, g, b, s, q):
    a, c = _affine(g, b, s, q)
    x = jnp.maximum(y * a[None, :] + c[None, :], 0.0)
    return lax.dot_general(x.astype(jnp.bfloat16), w.astype(jnp.bfloat16),
                           (((1,), (1,)), ((), ())),
                           preferred_element_type=jnp.float32)


def _stats1_body(gg_ref, ht_ref, sum_ref, sq_ref):
    _acc_stats(sum_ref, sq_ref, gg_ref[0] + ht_ref[...])


def _stats1(gg, ht):
    return pl.pallas_call(
        _stats1_body,
        grid=(P // _PB, NS),
        in_specs=[_GG_SPEC, _HT_SPEC],
        out_specs=_STATS_OUT_SPECS,
        out_shape=_STATS_OUT_SHAPE,
        interpret=_INTERP,
    )(gg.reshape(NS, P, C), ht)


def _stats2_body(gg_ref, ht_ref, w1_ref, g0_ref, b0_ref, s1_ref, q1_ref,
                 sum_ref, sq_ref):
    y2 = _bn_relu_mm(gg_ref[0] + ht_ref[...], w1_ref[...], g0_ref[0, :],
                     b0_ref[0, :], s1_ref[0, :], q1_ref[0, :])
    _acc_stats(sum_ref, sq_ref, y2)


def _stats2(gg, ht, w1, g0, b0, s1, q1):
    return pl.pallas_call(
        _stats2_body,
        grid=(P // _PB, NS),
        in_specs=[_GG_SPEC, _HT_SPEC, _W_SPEC] + [_V_SPEC] * 4,
        out_specs=_STATS_OUT_SPECS,
        out_shape=_STATS_OUT_SHAPE,
        interpret=_INTERP,
    )(gg.reshape(NS, P, C), ht, w1, g0, b0, s1, q1)


def _stats3_body(gg_ref, ht_ref, w1_ref, w2_ref, g0_ref, b0_ref, s1_ref,
                 q1_ref, g1_ref, b1_ref, s2_ref, q2_ref, sum_ref, sq_ref):
    y2 = _bn_relu_mm(gg_ref[0] + ht_ref[...], w1_ref[...], g0_ref[0, :],
                     b0_ref[0, :], s1_ref[0, :], q1_ref[0, :])
    y3 = _bn_relu_mm(y2, w2_ref[...], g1_ref[0, :], b1_ref[0, :],
                     s2_ref[0, :], q2_ref[0, :])
    _acc_stats(sum_ref, sq_ref, y3)


def _stats3(gg, ht, w1, w2, g0, b0, s1, q1, g1, b1, s2, q2):
    return pl.pallas_call(
        _stats3_body,
        grid=(P // _PB, NS),
        in_specs=[_GG_SPEC, _HT_SPEC, _W_SPEC, _W_SPEC] + [_V_SPEC] * 8,
        out_specs=_STATS_OUT_SPECS,
        out_shape=_STATS_OUT_SHAPE,
        interpret=_INTERP,
    )(gg.reshape(NS, P, C), ht, w1, w2, g0, b0, s1, q1, g1, b1, s2, q2)


_PB2 = 512


def _final_body(gg_ref, ht_ref, w1_ref, w2_ref, g0_ref, b0_ref, s1_ref,
                q1_ref, g1_ref, b1_ref, s2_ref, q2_ref, g2_ref, b2_ref,
                s3_ref, q3_ref, out_ref):
    y1 = (gg_ref[...] + ht_ref[...][None, :, :]).reshape(NS * _PB2, C)
    y2 = _bn_relu_mm(y1, w1_ref[...], g0_ref[0, :], b0_ref[0, :],
                     s1_ref[0, :], q1_ref[0, :])
    y3 = _bn_relu_mm(y2, w2_ref[...], g1_ref[0, :], b1_ref[0, :],
                     s2_ref[0, :], q2_ref[0, :])
    a, c = _affine(g2_ref[0, :], b2_ref[0, :], s3_ref[0, :], q3_ref[0, :])
    x = jnp.maximum(y3 * a[None, :] + c[None, :], 0.0)
    r = jnp.max(x.reshape(NS, _PB2, C), axis=0)
    out_ref[0] = r.T


def _final(gg, ht, w1, w2, g0, b0, s1, q1, g1, b1, s2, q2, g2, b2, s3, q3):
    nb = N // _PB2
    v = pl.BlockSpec((1, C), lambda t: (0, 0))
    w = pl.BlockSpec((C, C), lambda t: (0, 0))
    return pl.pallas_call(
        _final_body,
        grid=(P // _PB2,),
        in_specs=[
            pl.BlockSpec((NS, _PB2, C), lambda t: (0, t, 0)),
            pl.BlockSpec((_PB2, C), lambda t: (t, 0)),
            w, w, v, v, v, v, v, v, v, v, v, v, v, v,
        ],
        out_specs=pl.BlockSpec((1, C, _PB2), lambda t: (t // nb, 0, t % nb)),
        out_shape=jax.ShapeDtypeStruct((B, C, N), jnp.float32),
        interpret=_INTERP,
    )(gg.reshape(NS, P, C), ht, w1, w2, g0, b0, s1, q1, g1, b1, s2, q2,
      g2, b2, s3, q3)


def kernel(pos1, pos2, feature1, feature2, W0, W1, W2, g0, b0, g1, b1, g2, b2):
    wp = W0[:, :3]
    wf2 = W0[:, 3:3 + C]
    wf1 = W0[:, 3 + C:]
    r = lambda v: v.reshape(1, C)
    g0r, b0r, g1r, b1r, g2r, b2r = r(g0), r(b0), r(g1), r(b1), r(g2), r(b2)

    gt, ht = _proj(pos1, pos2, feature1, feature2, wp, wf1, wf2)
    idxf = _knn(pos1, pos2)
    gg = _gather_sc(gt, idxf.reshape(M // _CHUNK, _CHUNK))
    s1, q1 = _stats1(gg, ht)
    s2, q2 = _stats2(gg, ht, W1, g0r, b0r, s1, q1)
    s3, q3 = _stats3(gg, ht, W1, W2, g0r, b0r, s1, q1, g1r, b1r, s2, q2)
    feat = _final(gg, ht, W1, W2, g0r, b0r, s1, q1, g1r, b1r, s2, q2,
                  g2r, b2r, s3, q3)
    return (pos1, feat)


# speedup vs baseline: 1.1883x; 1.1883x over previous
import functools

import jax
import jax.numpy as jnp
from jax import lax
from jax.experimental import pallas as pl
from jax.experimental.pallas import tpu as pltpu
from jax.experimental.pallas import tpu_sc as plsc

B, N, C, NS = 4, 2048, 128, 16
C2 = C // 2
P = B * N
M = NS * P
EPS = 1e-5

_INTERP = False


def _pack_bf16(g):
    u = lax.bitcast_convert_type(g, jnp.uint32)
    r = (u + 0x7FFF + ((u >> 16) & 1)) >> 16
    lo = r[:, :C2]
    hi = r[:, C2:]
    return lax.bitcast_convert_type(lo | (hi << 16), jnp.float32)


def _unpack_bf16(gp):
    u = lax.bitcast_convert_type(gp, jnp.uint32)
    lo = lax.bitcast_convert_type(u << 16, jnp.float32)
    hi = lax.bitcast_convert_type(u & jnp.uint32(0xFFFF0000), jnp.float32)
    return jnp.concatenate([lo, hi], axis=-1)


def _proj_body(pos1_ref, pos2_ref, f1_ref, f2_ref, wp_ref, wf1_ref, wf2_ref,
               g_ref, h_ref):
    dn = (((0,), (1,)), ((), ()))
    g = lax.dot_general(f2_ref[0], wf2_ref[...], dn,
                        preferred_element_type=jnp.float32)
    g += lax.dot_general(pos2_ref[0], wp_ref[...], dn,
                         preferred_element_type=jnp.float32)
    g_ref[...] = g
    h = lax.dot_general(f1_ref[0], wf1_ref[...], dn,
                        preferred_element_type=jnp.float32)
    h -= lax.dot_general(pos1_ref[0], wp_ref[...], dn,
                         preferred_element_type=jnp.float32)
    h_ref[...] = h


def _proj(pos1, pos2, f1, f2, wp, wf1, wf2):
    return pl.pallas_call(
        _proj_body,
        grid=(B,),
        in_specs=[
            pl.BlockSpec((1, 3, N), lambda b: (b, 0, 0)),
            pl.BlockSpec((1, 3, N), lambda b: (b, 0, 0)),
            pl.BlockSpec((1, C, N), lambda b: (b, 0, 0)),
            pl.BlockSpec((1, C, N), lambda b: (b, 0, 0)),
            pl.BlockSpec((C, 3), lambda b: (0, 0)),
            pl.BlockSpec((C, C), lambda b: (0, 0)),
            pl.BlockSpec((C, C), lambda b: (0, 0)),
        ],
        out_specs=[
            pl.BlockSpec((N, C), lambda b: (b, 0)),
            pl.BlockSpec((N, C), lambda b: (b, 0)),
        ],
        out_shape=[
            jax.ShapeDtypeStruct((P, C), jnp.float32),
            jax.ShapeDtypeStruct((P, C), jnp.float32),
        ],
        interpret=_INTERP,
    )(pos1, pos2, f1, f2, wp, wf1, wf2)


_RB = 512


def _knn_body(p1_ref, p2_ref, out_ref):
    b = pl.program_id(0)
    p1 = p1_ref[0]
    p2 = p2_ref[0]
    d = -2.0 * lax.dot_general(p1, p2, (((0,), (0,)), ((), ())),
                               preferred_element_type=jnp.float32)
    d += jnp.sum(p2 * p2, axis=0)[None, :]
    d = d.T
    iota = lax.broadcasted_iota(jnp.int32, (N, _RB), 0)
    inf = jnp.float32(jnp.inf)
    for s in range(NS):
        am = jnp.argmin(d, axis=0).astype(jnp.int32)
        out_ref[s, :] = am + b * N
        d = jnp.where(iota == am[None, :], inf, d)


def _knn(pos1, pos2):
    return pl.pallas_call(
        _knn_body,
        grid=(B, N // _RB),
        in_specs=[
            pl.BlockSpec((1, 3, _RB), lambda b, i: (b, 0, i)),
            pl.BlockSpec((1, 3, N), lambda b, i: (b, 0, 0)),
        ],
        out_specs=pl.BlockSpec((NS, _RB), lambda b, i: (0, b * (N // _RB) + i)),
        out_shape=jax.ShapeDtypeStruct((NS, P), jnp.int32),
        interpret=_INTERP,
    )(pos1, pos2)


_NC_SC, _NSUB_SC = 2, 16
_NW = _NC_SC * _NSUB_SC
_ROWS_W = M // _NW
_CHUNK = 128
_NCHUNK = _ROWS_W // _CHUNK


def _gather_sc(table, idx2d):
    mesh = plsc.VectorSubcoreMesh(core_axis_name="c", subcore_axis_name="s")

    @functools.partial(
        pl.kernel, mesh=mesh,
        out_type=jax.ShapeDtypeStruct((M, C), jnp.float32),
        scratch_types=[
            pltpu.VMEM((_NCHUNK, _CHUNK), jnp.int32),
            pltpu.VMEM((_CHUNK, C), jnp.float32),
            pltpu.VMEM((_CHUNK, C), jnp.float32),
            pltpu.SemaphoreType.DMA,
            pltpu.SemaphoreType.DMA,
        ],
    )
    def k(table_hbm, idx_hbm, out_hbm, idx_v, buf0, buf1, sem0, sem1):
        wid = lax.axis_index("s") * _NC_SC + lax.axis_index("c")
        pltpu.sync_copy(idx_hbm.at[pl.ds(wid * _NCHUNK, _NCHUNK)], idx_v)
        out_base = wid * _ROWS_W

        def body(j2, _):
            j0 = j2 * 2
            cp0 = pltpu.async_copy(table_hbm.at[idx_v.at[j0]], buf0, sem0)
            cp1 = pltpu.async_copy(table_hbm.at[idx_v.at[j0 + 1]], buf1, sem1)
            cp0.wait()
            pltpu.sync_copy(buf0, out_hbm.at[pl.ds(out_base + j0 * _CHUNK,
                                                   _CHUNK)])
            cp1.wait()
            pltpu.sync_copy(buf1, out_hbm.at[pl.ds(out_base + (j0 + 1) * _CHUNK,
                                                   _CHUNK)])
            return 0

        lax.fori_loop(0, _NCHUNK // 2, body, 0)

    return k(table, idx2d)


def _affine(g, b, s, q):
    mean = s / jnp.float32(M)
    var = q / jnp.float32(M) - mean * mean
    a = g * lax.rsqrt(var + EPS)
    c = b - mean * a
    return a, c


_PB = 2048

_STATS_OUT_SPECS = [
    pl.BlockSpec((1, C), lambda i, s: (0, 0)),
    pl.BlockSpec((1, C), lambda i, s: (0, 0)),
]
_STATS_OUT_SHAPE = [
    jax.ShapeDtypeStruct((1, C), jnp.float32),
    jax.ShapeDtypeStruct((1, C), jnp.float32),
]
_GG_SPEC = pl.BlockSpec((1, _PB, C), lambda i, s: (s, i, 0))
_GGP_SPEC = pl.BlockSpec((1, _PB, C2), lambda i, s: (s, i, 0))
_HT_SPEC = pl.BlockSpec((_PB, C), lambda i, s: (i, 0))
_W_SPEC = pl.BlockSpec((C, C), lambda i, s: (0, 0))
_V_SPEC = pl.BlockSpec((1, C), lambda i, s: (0, 0))


def _acc_stats(sum_ref, sq_ref, y):
    @pl.when((pl.program_id(0) == 0) & (pl.program_id(1) == 0))
    def _():
        sum_ref[...] = jnp.zeros_like(sum_ref)
        sq_ref[...] = jnp.zeros_like(sq_ref)

    sum_ref[0, :] += jnp.sum(y, axis=0)
    sq_ref[0, :] += jnp.sum(y * y, axis=0)


def _bn_relu_mm(y, w, g, b, s, q):
    a, c = _affine(g, b, s, q)
    x = jnp.maximum(y * a[None, :] + c[None, :], 0.0)
    return lax.dot_general(x, w, (((1,), (1,)), ((), ())),
                           preferred_element_type=jnp.float32)


def _stats1_body(gg_ref, ht_ref, sum_ref, sq_ref):
    _acc_stats(sum_ref, sq_ref, gg_ref[0] + ht_ref[...])


def _stats1(gg, ht):
    return pl.pallas_call(
        _stats1_body,
        grid=(P // _PB, NS),
        in_specs=[_GG_SPEC, _HT_SPEC],
        out_specs=_STATS_OUT_SPECS,
        out_shape=_STATS_OUT_SHAPE,
        interpret=_INTERP,
    )(gg.reshape(NS, P, C), ht)


def _stats2_body(gg_ref, ht_ref, w1_ref, g0_ref, b0_ref, s1_ref, q1_ref,
                 sum_ref, sq_ref):
    y2 = _bn_relu_mm(gg_ref[0] + ht_ref[...], w1_ref[...], g0_ref[0, :],
                     b0_ref[0, :], s1_ref[0, :], q1_ref[0, :])
    _acc_stats(sum_ref, sq_ref, y2)


def _stats2(gg, ht, w1, g0, b0, s1, q1):
    return pl.pallas_call(
        _stats2_body,
        grid=(P // _PB, NS),
        in_specs=[_GG_SPEC, _HT_SPEC, _W_SPEC] + [_V_SPEC] * 4,
        out_specs=_STATS_OUT_SPECS,
        out_shape=_STATS_OUT_SHAPE,
        interpret=_INTERP,
    )(gg.reshape(NS, P, C), ht, w1, g0, b0, s1, q1)


def _stats3_body(gg_ref, ht_ref, w1_ref, w2_ref, g0_ref, b0_ref, s1_ref,
                 q1_ref, g1_ref, b1_ref, s2_ref, q2_ref, sum_ref, sq_ref):
    y2 = _bn_relu_mm(gg_ref[0] + ht_ref[...], w1_ref[...], g0_ref[0, :],
                     b0_ref[0, :], s1_ref[0, :], q1_ref[0, :])
    y3 = _bn_relu_mm(y2, w2_ref[...], g1_ref[0, :], b1_ref[0, :],
                     s2_ref[0, :], q2_ref[0, :])
    _acc_stats(sum_ref, sq_ref, y3)


def _stats3(gg, ht, w1, w2, g0, b0, s1, q1, g1, b1, s2, q2):
    return pl.pallas_call(
        _stats3_body,
        grid=(P // _PB, NS),
        in_specs=[_GG_SPEC, _HT_SPEC, _W_SPEC, _W_SPEC] + [_V_SPEC] * 8,
        out_specs=_STATS_OUT_SPECS,
        out_shape=_STATS_OUT_SHAPE,
        interpret=_INTERP,
    )(gg.reshape(NS, P, C), ht, w1, w2, g0, b0, s1, q1, g1, b1, s2, q2)


_PB2 = 512


def _final_body(gg_ref, ht_ref, w1_ref, w2_ref, g0_ref, b0_ref, s1_ref,
                q1_ref, g1_ref, b1_ref, s2_ref, q2_ref, g2_ref, b2_ref,
                s3_ref, q3_ref, out_ref):
    y1 = (gg_ref[...] + ht_ref[...][None, :, :]).reshape(NS * _PB2, C)
    y2 = _bn_relu_mm(y1, w1_ref[...], g0_ref[0, :], b0_ref[0, :],
                     s1_ref[0, :], q1_ref[0, :])
    y3 = _bn_relu_mm(y2, w2_ref[...], g1_ref[0, :], b1_ref[0, :],
                     s2_ref[0, :], q2_ref[0, :])
    a, c = _affine(g2_ref[0, :], b2_ref[0, :], s3_ref[0, :], q3_ref[0, :])
    x = jnp.maximum(y3 * a[None, :] + c[None, :], 0.0)
    r = jnp.max(x.reshape(NS, _PB2, C), axis=0)
    out_ref[0] = r.T


def _final(gg, ht, w1, w2, g0, b0, s1, q1, g1, b1, s2, q2, g2, b2, s3, q3):
    nb = N // _PB2
    v = pl.BlockSpec((1, C), lambda t: (0, 0))
    w = pl.BlockSpec((C, C), lambda t: (0, 0))
    return pl.pallas_call(
        _final_body,
        grid=(P // _PB2,),
        in_specs=[
            pl.BlockSpec((NS, _PB2, C), lambda t: (0, t, 0)),
            pl.BlockSpec((_PB2, C), lambda t: (t, 0)),
            w, w, v, v, v, v, v, v, v, v, v, v, v, v,
        ],
        out_specs=pl.BlockSpec((1, C, _PB2), lambda t: (t // nb, 0, t % nb)),
        out_shape=jax.ShapeDtypeStruct((B, C, N), jnp.float32),
        interpret=_INTERP,
    )(gg.reshape(NS, P, C), ht, w1, w2, g0, b0, s1, q1, g1, b1, s2, q2,
      g2, b2, s3, q3)


def kernel(pos1, pos2, feature1, feature2, W0, W1, W2, g0, b0, g1, b1, g2, b2):
    wp = W0[:, :3]
    wf2 = W0[:, 3:3 + C]
    wf1 = W0[:, 3 + C:]
    r = lambda v: v.reshape(1, C)
    g0r, b0r, g1r, b1r, g2r, b2r = r(g0), r(b0), r(g1), r(b1), r(g2), r(b2)

    gt, ht = _proj(pos1, pos2, feature1, feature2, wp, wf1, wf2)
    idxf = _knn(pos1, pos2)
    gg = _gather_sc(gt, idxf.reshape(M // _CHUNK, _CHUNK))
    s1, q1 = _stats1(gg, ht)
    s2, q2 = _stats2(gg, ht, W1, g0r, b0r, s1, q1)
    s3, q3 = _stats3(gg, ht, W1, W2, g0r, b0r, s1, q1, g1r, b1r, s2, q2)
    feat = _final(gg, ht, W1, W2, g0r, b0r, s1, q1, g1r, b1r, s2, q2,
                  g2r, b2r, s3, q3)
    return (pos1, feat)


# speedup vs baseline: 1.3270x; 1.1166x over previous
import functools

import jax
import jax.numpy as jnp
from jax import lax
from jax.experimental import pallas as pl
from jax.experimental.pallas import tpu as pltpu
from jax.experimental.pallas import tpu_sc as plsc

B, N, C, NS = 4, 2048, 128, 16
C2 = C // 2
P = B * N
M = NS * P
EPS = 1e-5

_INTERP = False


def _pack_bf16(g):
    u = lax.bitcast_convert_type(g, jnp.uint32)
    r = (u + 0x7FFF + ((u >> 16) & 1)) >> 16
    lo = r[:, :C2]
    hi = r[:, C2:]
    return lax.bitcast_convert_type(lo | (hi << 16), jnp.float32)


def _unpack_bf16(gp):
    u = lax.bitcast_convert_type(gp, jnp.uint32)
    lo = lax.bitcast_convert_type(u << 16, jnp.float32)
    hi = lax.bitcast_convert_type(u & jnp.uint32(0xFFFF0000), jnp.float32)
    return jnp.concatenate([lo, hi], axis=-1)


def _proj_body(pos1_ref, pos2_ref, f1_ref, f2_ref, wp_ref, wf1_ref, wf2_ref,
               g_ref, h_ref):
    dn = (((0,), (1,)), ((), ()))
    g = lax.dot_general(f2_ref[0], wf2_ref[...], dn,
                        preferred_element_type=jnp.float32)
    g += lax.dot_general(pos2_ref[0], wp_ref[...], dn,
                         preferred_element_type=jnp.float32)
    g_ref[...] = g
    h = lax.dot_general(f1_ref[0], wf1_ref[...], dn,
                        preferred_element_type=jnp.float32)
    h -= lax.dot_general(pos1_ref[0], wp_ref[...], dn,
                         preferred_element_type=jnp.float32)
    h_ref[...] = h


def _proj(pos1, pos2, f1, f2, wp, wf1, wf2):
    return pl.pallas_call(
        _proj_body,
        grid=(B,),
        in_specs=[
            pl.BlockSpec((1, 3, N), lambda b: (b, 0, 0)),
            pl.BlockSpec((1, 3, N), lambda b: (b, 0, 0)),
            pl.BlockSpec((1, C, N), lambda b: (b, 0, 0)),
            pl.BlockSpec((1, C, N), lambda b: (b, 0, 0)),
            pl.BlockSpec((C, 3), lambda b: (0, 0)),
            pl.BlockSpec((C, C), lambda b: (0, 0)),
            pl.BlockSpec((C, C), lambda b: (0, 0)),
        ],
        out_specs=[
            pl.BlockSpec((N, C), lambda b: (b, 0)),
            pl.BlockSpec((N, C), lambda b: (b, 0)),
        ],
        out_shape=[
            jax.ShapeDtypeStruct((P, C), jnp.float32),
            jax.ShapeDtypeStruct((P, C), jnp.float32),
        ],
        interpret=_INTERP,
    )(pos1, pos2, f1, f2, wp, wf1, wf2)


_RB = 512


def _knn_body(p1_ref, p2_ref, out_ref):
    b = pl.program_id(0)
    p1 = p1_ref[0]
    p2 = p2_ref[0]
    d = -2.0 * lax.dot_general(p1, p2, (((0,), (0,)), ((), ())),
                               preferred_element_type=jnp.float32)
    d += jnp.sum(p2 * p2, axis=0)[None, :]
    d = d.T
    iota = lax.broadcasted_iota(jnp.int32, (N, _RB), 0)
    inf = jnp.float32(jnp.inf)
    for s in range(NS):
        am = jnp.argmin(d, axis=0).astype(jnp.int32)
        out_ref[s, :] = am + b * N
        d = jnp.where(iota == am[None, :], inf, d)


def _knn(pos1, pos2):
    return pl.pallas_call(
        _knn_body,
        grid=(B, N // _RB),
        in_specs=[
            pl.BlockSpec((1, 3, _RB), lambda b, i: (b, 0, i)),
            pl.BlockSpec((1, 3, N), lambda b, i: (b, 0, 0)),
        ],
        out_specs=pl.BlockSpec((NS, _RB), lambda b, i: (0, b * (N // _RB) + i)),
        out_shape=jax.ShapeDtypeStruct((NS, P), jnp.int32),
        interpret=_INTERP,
    )(pos1, pos2)


_NC_SC, _NSUB_SC = 2, 16
_NW = _NC_SC * _NSUB_SC
_ROWS_W = M // _NW
_CHUNK = 128
_NCHUNK = _ROWS_W // _CHUNK


def _gather_sc(table, idx2d):
    mesh = plsc.VectorSubcoreMesh(core_axis_name="c", subcore_axis_name="s")

    @functools.partial(
        pl.kernel, mesh=mesh,
        out_type=jax.ShapeDtypeStruct((M, C), jnp.float32),
        scratch_types=[
            pltpu.VMEM((_NCHUNK, _CHUNK), jnp.int32),
            pltpu.VMEM((_CHUNK, C), jnp.float32),
            pltpu.VMEM((_CHUNK, C), jnp.float32),
            pltpu.SemaphoreType.DMA,
            pltpu.SemaphoreType.DMA,
        ],
    )
    def k(table_hbm, idx_hbm, out_hbm, idx_v, buf0, buf1, sem0, sem1):
        wid = lax.axis_index("s") * _NC_SC + lax.axis_index("c")
        pltpu.sync_copy(idx_hbm.at[pl.ds(wid * _NCHUNK, _NCHUNK)], idx_v)
        out_base = wid * _ROWS_W

        def body(j2, _):
            j0 = j2 * 2
            cp0 = pltpu.async_copy(table_hbm.at[idx_v.at[j0]], buf0, sem0)
            cp1 = pltpu.async_copy(table_hbm.at[idx_v.at[j0 + 1]], buf1, sem1)
            cp0.wait()
            pltpu.sync_copy(buf0, out_hbm.at[pl.ds(out_base + j0 * _CHUNK,
                                                   _CHUNK)])
            cp1.wait()
            pltpu.sync_copy(buf1, out_hbm.at[pl.ds(out_base + (j0 + 1) * _CHUNK,
                                                   _CHUNK)])
            return 0

        lax.fori_loop(0, _NCHUNK // 2, body, 0)

    return k(table, idx2d)


def _affine(g, b, s, q):
    mean = s / jnp.float32(M)
    var = q / jnp.float32(M) - mean * mean
    a = g * lax.rsqrt(var + EPS)
    c = b - mean * a
    return a, c


_PB = 4096

_STATS_OUT_SPECS = [
    pl.BlockSpec((1, C), lambda i, s: (0, 0)),
    pl.BlockSpec((1, C), lambda i, s: (0, 0)),
]
_STATS_OUT_SHAPE = [
    jax.ShapeDtypeStruct((1, C), jnp.float32),
    jax.ShapeDtypeStruct((1, C), jnp.float32),
]
_GG_SPEC = pl.BlockSpec((1, _PB, C), lambda i, s: (s, i, 0))
_GGP_SPEC = pl.BlockSpec((1, _PB, C2), lambda i, s: (s, i, 0))
_HT_SPEC = pl.BlockSpec((_PB, C), lambda i, s: (i, 0))
_W_SPEC = pl.BlockSpec((C, C), lambda i, s: (0, 0))
_V_SPEC = pl.BlockSpec((1, C), lambda i, s: (0, 0))


def _acc_stats(sum_ref, sq_ref, y):
    @pl.when((pl.program_id(0) == 0) & (pl.program_id(1) == 0))
    def _():
        sum_ref[...] = jnp.zeros_like(sum_ref)
        sq_ref[...] = jnp.zeros_like(sq_ref)

    sum_ref[0, :] += jnp.sum(y, axis=0)
    sq_ref[0, :] += jnp.sum(y * y, axis=0)


def _bn_relu_mm(y, w, g, b, s, q):
    a, c = _affine(g, b, s, q)
    x = jnp.maximum(y * a[None, :] + c[None, :], 0.0)
    return lax.dot_general(x, w, (((1,), (1,)), ((), ())),
                           preferred_element_type=jnp.float32)


def _stats1_body(gg_ref, ht_ref, sum_ref, sq_ref):
    _acc_stats(sum_ref, sq_ref, gg_ref[0] + ht_ref[...])


def _stats1(gg, ht):
    return pl.pallas_call(
        _stats1_body,
        grid=(P // _PB, NS),
        in_specs=[_GG_SPEC, _HT_SPEC],
        out_specs=_STATS_OUT_SPECS,
        out_shape=_STATS_OUT_SHAPE,
        interpret=_INTERP,
    )(gg.reshape(NS, P, C), ht)


def _stats2_body(gg_ref, ht_ref, w1_ref, g0_ref, b0_ref, s1_ref, q1_ref,
                 sum_ref, sq_ref):
    y2 = _bn_relu_mm(gg_ref[0] + ht_ref[...], w1_ref[...], g0_ref[0, :],
                     b0_ref[0, :], s1_ref[0, :], q1_ref[0, :])
    _acc_stats(sum_ref, sq_ref, y2)


def _stats2(gg, ht, w1, g0, b0, s1, q1):
    return pl.pallas_call(
        _stats2_body,
        grid=(P // _PB, NS),
        in_specs=[_GG_SPEC, _HT_SPEC, _W_SPEC] + [_V_SPEC] * 4,
        out_specs=_STATS_OUT_SPECS,
        out_shape=_STATS_OUT_SHAPE,
        interpret=_INTERP,
    )(gg.reshape(NS, P, C), ht, w1, g0, b0, s1, q1)


def _stats3_body(gg_ref, ht_ref, w1_ref, w2_ref, g0_ref, b0_ref, s1_ref,
                 q1_ref, g1_ref, b1_ref, s2_ref, q2_ref, sum_ref, sq_ref):
    y2 = _bn_relu_mm(gg_ref[0] + ht_ref[...], w1_ref[...], g0_ref[0, :],
                     b0_ref[0, :], s1_ref[0, :], q1_ref[0, :])
    y3 = _bn_relu_mm(y2, w2_ref[...], g1_ref[0, :], b1_ref[0, :],
                     s2_ref[0, :], q2_ref[0, :])
    _acc_stats(sum_ref, sq_ref, y3)


def _stats3(gg, ht, w1, w2, g0, b0, s1, q1, g1, b1, s2, q2):
    return pl.pallas_call(
        _stats3_body,
        grid=(P // _PB, NS),
        in_specs=[_GG_SPEC, _HT_SPEC, _W_SPEC, _W_SPEC] + [_V_SPEC] * 8,
        out_specs=_STATS_OUT_SPECS,
        out_shape=_STATS_OUT_SHAPE,
        interpret=_INTERP,
    )(gg.reshape(NS, P, C), ht, w1, w2, g0, b0, s1, q1, g1, b1, s2, q2)


_PB2 = 1024


def _final_body(gg_ref, ht_ref, w1_ref, w2_ref, g0_ref, b0_ref, s1_ref,
                q1_ref, g1_ref, b1_ref, s2_ref, q2_ref, g2_ref, b2_ref,
                s3_ref, q3_ref, out_ref):
    y1 = (gg_ref[...] + ht_ref[...][None, :, :]).reshape(NS * _PB2, C)
    y2 = _bn_relu_mm(y1, w1_ref[...], g0_ref[0, :], b0_ref[0, :],
                     s1_ref[0, :], q1_ref[0, :])
    y3 = _bn_relu_mm(y2, w2_ref[...], g1_ref[0, :], b1_ref[0, :],
                     s2_ref[0, :], q2_ref[0, :])
    a, c = _affine(g2_ref[0, :], b2_ref[0, :], s3_ref[0, :], q3_ref[0, :])
    x = jnp.maximum(y3 * a[None, :] + c[None, :], 0.0)
    r = jnp.max(x.reshape(NS, _PB2, C), axis=0)
    out_ref[0] = r.T


def _final(gg, ht, w1, w2, g0, b0, s1, q1, g1, b1, s2, q2, g2, b2, s3, q3):
    nb = N // _PB2
    v = pl.BlockSpec((1, C), lambda t: (0, 0))
    w = pl.BlockSpec((C, C), lambda t: (0, 0))
    return pl.pallas_call(
        _final_body,
        grid=(P // _PB2,),
        in_specs=[
            pl.BlockSpec((NS, _PB2, C), lambda t: (0, t, 0)),
            pl.BlockSpec((_PB2, C), lambda t: (t, 0)),
            w, w, v, v, v, v, v, v, v, v, v, v, v, v,
        ],
        out_specs=pl.BlockSpec((1, C, _PB2), lambda t: (t // nb, 0, t % nb)),
        out_shape=jax.ShapeDtypeStruct((B, C, N), jnp.float32),
        interpret=_INTERP,
    )(gg.reshape(NS, P, C), ht, w1, w2, g0, b0, s1, q1, g1, b1, s2, q2,
      g2, b2, s3, q3)


def kernel(pos1, pos2, feature1, feature2, W0, W1, W2, g0, b0, g1, b1, g2, b2):
    wp = W0[:, :3]
    wf2 = W0[:, 3:3 + C]
    wf1 = W0[:, 3 + C:]
    r = lambda v: v.reshape(1, C)
    g0r, b0r, g1r, b1r, g2r, b2r = r(g0), r(b0), r(g1), r(b1), r(g2), r(b2)

    gt, ht = _proj(pos1, pos2, feature1, feature2, wp, wf1, wf2)
    idxf = _knn(pos1, pos2)
    gg = _gather_sc(gt, idxf.reshape(M // _CHUNK, _CHUNK))
    s1, q1 = _stats1(gg, ht)
    s2, q2 = _stats2(gg, ht, W1, g0r, b0r, s1, q1)
    s3, q3 = _stats3(gg, ht, W1, W2, g0r, b0r, s1, q1, g1r, b1r, s2, q2)
    feat = _final(gg, ht, W1, W2, g0r, b0r, s1, q1, g1r, b1r, s2, q2,
                  g2r, b2r, s3, q3)
    return (pos1, feat)


# speedup vs baseline: 1.3940x; 1.0505x over previous
import functools

import jax
import jax.numpy as jnp
from jax import lax
from jax.experimental import pallas as pl
from jax.experimental.pallas import tpu as pltpu
from jax.experimental.pallas import tpu_sc as plsc

B, N, C, NS = 4, 2048, 128, 16
C2 = C // 2
P = B * N
M = NS * P
EPS = 1e-5

_INTERP = False


def _pack_bf16(g):
    u = lax.bitcast_convert_type(g, jnp.uint32)
    r = (u + 0x7FFF + ((u >> 16) & 1)) >> 16
    lo = r[:, :C2]
    hi = r[:, C2:]
    return lax.bitcast_convert_type(lo | (hi << 16), jnp.float32)


def _unpack_bf16(gp):
    u = lax.bitcast_convert_type(gp, jnp.uint32)
    lo = lax.bitcast_convert_type(u << 16, jnp.float32)
    hi = lax.bitcast_convert_type(u & jnp.uint32(0xFFFF0000), jnp.float32)
    return jnp.concatenate([lo, hi], axis=-1)


def _proj_body(pos1_ref, pos2_ref, f1_ref, f2_ref, wp_ref, wf1_ref, wf2_ref,
               g_ref, h_ref):
    dn = (((0,), (1,)), ((), ()))
    g = lax.dot_general(f2_ref[0], wf2_ref[...], dn,
                        preferred_element_type=jnp.float32)
    g += lax.dot_general(pos2_ref[0], wp_ref[...], dn,
                         preferred_element_type=jnp.float32)
    g_ref[...] = g
    h = lax.dot_general(f1_ref[0], wf1_ref[...], dn,
                        preferred_element_type=jnp.float32)
    h -= lax.dot_general(pos1_ref[0], wp_ref[...], dn,
                         preferred_element_type=jnp.float32)
    h_ref[...] = h


def _proj(pos1, pos2, f1, f2, wp, wf1, wf2):
    return pl.pallas_call(
        _proj_body,
        grid=(B,),
        in_specs=[
            pl.BlockSpec((1, 3, N), lambda b: (b, 0, 0)),
            pl.BlockSpec((1, 3, N), lambda b: (b, 0, 0)),
            pl.BlockSpec((1, C, N), lambda b: (b, 0, 0)),
            pl.BlockSpec((1, C, N), lambda b: (b, 0, 0)),
            pl.BlockSpec((C, 3), lambda b: (0, 0)),
            pl.BlockSpec((C, C), lambda b: (0, 0)),
            pl.BlockSpec((C, C), lambda b: (0, 0)),
        ],
        out_specs=[
            pl.BlockSpec((N, C), lambda b: (b, 0)),
            pl.BlockSpec((N, C), lambda b: (b, 0)),
        ],
        out_shape=[
            jax.ShapeDtypeStruct((P, C), jnp.float32),
            jax.ShapeDtypeStruct((P, C), jnp.float32),
        ],
        interpret=_INTERP,
    )(pos1, pos2, f1, f2, wp, wf1, wf2)


_RB = 512


def _knn_body(p1_ref, p2_ref, out_ref):
    b = pl.program_id(0)
    p1 = p1_ref[0]
    p2 = p2_ref[0]
    d = -2.0 * lax.dot_general(p1, p2, (((0,), (0,)), ((), ())),
                               preferred_element_type=jnp.float32)
    d += jnp.sum(p2 * p2, axis=0)[None, :]
    d = d.T
    iota = lax.broadcasted_iota(jnp.int32, (N, _RB), 0)
    inf = jnp.float32(jnp.inf)
    for s in range(NS):
        am = jnp.argmin(d, axis=0).astype(jnp.int32)
        out_ref[s, :] = am + b * N
        d = jnp.where(iota == am[None, :], inf, d)


def _knn(pos1, pos2):
    return pl.pallas_call(
        _knn_body,
        grid=(B, N // _RB),
        in_specs=[
            pl.BlockSpec((1, 3, _RB), lambda b, i: (b, 0, i)),
            pl.BlockSpec((1, 3, N), lambda b, i: (b, 0, 0)),
        ],
        out_specs=pl.BlockSpec((NS, _RB), lambda b, i: (0, b * (N // _RB) + i)),
        out_shape=jax.ShapeDtypeStruct((NS, P), jnp.int32),
        interpret=_INTERP,
    )(pos1, pos2)


_NC_SC, _NSUB_SC = 2, 16
_NW = _NC_SC * _NSUB_SC
_ROWS_W = M // _NW
_CHUNK = 128
_NCHUNK = _ROWS_W // _CHUNK


def _gather_sc(table, idx2d):
    mesh = plsc.VectorSubcoreMesh(core_axis_name="c", subcore_axis_name="s")

    @functools.partial(
        pl.kernel, mesh=mesh,
        out_type=jax.ShapeDtypeStruct((M, C), jnp.float32),
        scratch_types=[
            pltpu.VMEM((_NCHUNK, _CHUNK), jnp.int32),
            pltpu.VMEM((_CHUNK, C), jnp.float32),
            pltpu.VMEM((_CHUNK, C), jnp.float32),
            pltpu.SemaphoreType.DMA,
            pltpu.SemaphoreType.DMA,
        ],
    )
    def k(table_hbm, idx_hbm, out_hbm, idx_v, buf0, buf1, sem0, sem1):
        wid = lax.axis_index("s") * _NC_SC + lax.axis_index("c")
        pltpu.sync_copy(idx_hbm.at[pl.ds(wid * _NCHUNK, _NCHUNK)], idx_v)
        out_base = wid * _ROWS_W

        def body(j2, _):
            j0 = j2 * 2
            cp0 = pltpu.async_copy(table_hbm.at[idx_v.at[j0]], buf0, sem0)
            cp1 = pltpu.async_copy(table_hbm.at[idx_v.at[j0 + 1]], buf1, sem1)
            cp0.wait()
            pltpu.sync_copy(buf0, out_hbm.at[pl.ds(out_base + j0 * _CHUNK,
                                                   _CHUNK)])
            cp1.wait()
            pltpu.sync_copy(buf1, out_hbm.at[pl.ds(out_base + (j0 + 1) * _CHUNK,
                                                   _CHUNK)])
            return 0

        lax.fori_loop(0, _NCHUNK // 2, body, 0)

    return k(table, idx2d)


def _affine(g, b, s, q):
    mean = s / jnp.float32(M)
    var = q / jnp.float32(M) - mean * mean
    a = g * lax.rsqrt(var + EPS)
    c = b - mean * a
    return a, c


_PB = 8192

_STATS_OUT_SPECS = [
    pl.BlockSpec((1, C), lambda i, s: (0, 0)),
    pl.BlockSpec((1, C), lambda i, s: (0, 0)),
]
_STATS_OUT_SHAPE = [
    jax.ShapeDtypeStruct((1, C), jnp.float32),
    jax.ShapeDtypeStruct((1, C), jnp.float32),
]
_GG_SPEC = pl.BlockSpec((1, _PB, C), lambda i, s: (s, i, 0))
_GGP_SPEC = pl.BlockSpec((1, _PB, C2), lambda i, s: (s, i, 0))
_HT_SPEC = pl.BlockSpec((_PB, C), lambda i, s: (i, 0))
_W_SPEC = pl.BlockSpec((C, C), lambda i, s: (0, 0))
_V_SPEC = pl.BlockSpec((1, C), lambda i, s: (0, 0))


def _acc_stats(sum_ref, sq_ref, y):
    @pl.when((pl.program_id(0) == 0) & (pl.program_id(1) == 0))
    def _():
        sum_ref[...] = jnp.zeros_like(sum_ref)
        sq_ref[...] = jnp.zeros_like(sq_ref)

    sum_ref[0, :] += jnp.sum(y, axis=0)
    sq_ref[0, :] += jnp.sum(y * y, axis=0)


def _bn_relu_mm(y, w, g, b, s, q):
    a, c = _affine(g, b, s, q)
    x = jnp.maximum(y * a[None, :] + c[None, :], 0.0)
    return lax.dot_general(x, w, (((1,), (1,)), ((), ())),
                           preferred_element_type=jnp.float32)


def _stats1_body(gg_ref, ht_ref, sum_ref, sq_ref):
    _acc_stats(sum_ref, sq_ref, gg_ref[0] + ht_ref[...])


def _stats1(gg, ht):
    return pl.pallas_call(
        _stats1_body,
        grid=(P // _PB, NS),
        in_specs=[_GG_SPEC, _HT_SPEC],
        out_specs=_STATS_OUT_SPECS,
        out_shape=_STATS_OUT_SHAPE,
        interpret=_INTERP,
    )(gg.reshape(NS, P, C), ht)


def _stats2_body(gg_ref, ht_ref, w1_ref, g0_ref, b0_ref, s1_ref, q1_ref,
                 sum_ref, sq_ref):
    y2 = _bn_relu_mm(gg_ref[0] + ht_ref[...], w1_ref[...], g0_ref[0, :],
                     b0_ref[0, :], s1_ref[0, :], q1_ref[0, :])
    _acc_stats(sum_ref, sq_ref, y2)


def _stats2(gg, ht, w1, g0, b0, s1, q1):
    return pl.pallas_call(
        _stats2_body,
        grid=(P // _PB, NS),
        in_specs=[_GG_SPEC, _HT_SPEC, _W_SPEC] + [_V_SPEC] * 4,
        out_specs=_STATS_OUT_SPECS,
        out_shape=_STATS_OUT_SHAPE,
        interpret=_INTERP,
    )(gg.reshape(NS, P, C), ht, w1, g0, b0, s1, q1)


def _stats3_body(gg_ref, ht_ref, w1_ref, w2_ref, g0_ref, b0_ref, s1_ref,
                 q1_ref, g1_ref, b1_ref, s2_ref, q2_ref, sum_ref, sq_ref):
    y2 = _bn_relu_mm(gg_ref[0] + ht_ref[...], w1_ref[...], g0_ref[0, :],
                     b0_ref[0, :], s1_ref[0, :], q1_ref[0, :])
    y3 = _bn_relu_mm(y2, w2_ref[...], g1_ref[0, :], b1_ref[0, :],
                     s2_ref[0, :], q2_ref[0, :])
    _acc_stats(sum_ref, sq_ref, y3)


def _stats3(gg, ht, w1, w2, g0, b0, s1, q1, g1, b1, s2, q2):
    return pl.pallas_call(
        _stats3_body,
        grid=(P // _PB, NS),
        in_specs=[_GG_SPEC, _HT_SPEC, _W_SPEC, _W_SPEC] + [_V_SPEC] * 8,
        out_specs=_STATS_OUT_SPECS,
        out_shape=_STATS_OUT_SHAPE,
        interpret=_INTERP,
    )(gg.reshape(NS, P, C), ht, w1, w2, g0, b0, s1, q1, g1, b1, s2, q2)


_PB2 = 2048


def _final_body(gg_ref, ht_ref, w1_ref, w2_ref, g0_ref, b0_ref, s1_ref,
                q1_ref, g1_ref, b1_ref, s2_ref, q2_ref, g2_ref, b2_ref,
                s3_ref, q3_ref, out_ref):
    y1 = (gg_ref[...] + ht_ref[...][None, :, :]).reshape(NS * _PB2, C)
    y2 = _bn_relu_mm(y1, w1_ref[...], g0_ref[0, :], b0_ref[0, :],
                     s1_ref[0, :], q1_ref[0, :])
    y3 = _bn_relu_mm(y2, w2_ref[...], g1_ref[0, :], b1_ref[0, :],
                     s2_ref[0, :], q2_ref[0, :])
    a, c = _affine(g2_ref[0, :], b2_ref[0, :], s3_ref[0, :], q3_ref[0, :])
    x = jnp.maximum(y3 * a[None, :] + c[None, :], 0.0)
    r = jnp.max(x.reshape(NS, _PB2, C), axis=0)
    out_ref[0] = r.T


def _final(gg, ht, w1, w2, g0, b0, s1, q1, g1, b1, s2, q2, g2, b2, s3, q3):
    nb = N // _PB2
    v = pl.BlockSpec((1, C), lambda t: (0, 0))
    w = pl.BlockSpec((C, C), lambda t: (0, 0))
    return pl.pallas_call(
        _final_body,
        grid=(P // _PB2,),
        in_specs=[
            pl.BlockSpec((NS, _PB2, C), lambda t: (0, t, 0)),
            pl.BlockSpec((_PB2, C), lambda t: (t, 0)),
            w, w, v, v, v, v, v, v, v, v, v, v, v, v,
        ],
        out_specs=pl.BlockSpec((1, C, _PB2), lambda t: (t // nb, 0, t % nb)),
        out_shape=jax.ShapeDtypeStruct((B, C, N), jnp.float32),
        interpret=_INTERP,
    )(gg.reshape(NS, P, C), ht, w1, w2, g0, b0, s1, q1, g1, b1, s2, q2,
      g2, b2, s3, q3)


def kernel(pos1, pos2, feature1, feature2, W0, W1, W2, g0, b0, g1, b1, g2, b2):
    wp = W0[:, :3]
    wf2 = W0[:, 3:3 + C]
    wf1 = W0[:, 3 + C:]
    r = lambda v: v.reshape(1, C)
    g0r, b0r, g1r, b1r, g2r, b2r = r(g0), r(b0), r(g1), r(b1), r(g2), r(b2)

    gt, ht = _proj(pos1, pos2, feature1, feature2, wp, wf1, wf2)
    idxf = _knn(pos1, pos2)
    gg = _gather_sc(gt, idxf.reshape(M // _CHUNK, _CHUNK))
    s1, q1 = _stats1(gg, ht)
    s2, q2 = _stats2(gg, ht, W1, g0r, b0r, s1, q1)
    s3, q3 = _stats3(gg, ht, W1, W2, g0r, b0r, s1, q1, g1r, b1r, s2, q2)
    feat = _final(gg, ht, W1, W2, g0r, b0r, s1, q1, g1r, b1r, s2, q2,
                  g2r, b2r, s3, q3)
    return (pos1, feat)


# speedup vs baseline: 1.4162x; 1.0159x over previous
import functools

import jax
import jax.numpy as jnp
from jax import lax
from jax.experimental import pallas as pl
from jax.experimental.pallas import tpu as pltpu
from jax.experimental.pallas import tpu_sc as plsc

B, N, C, NS = 4, 2048, 128, 16
C2 = C // 2
P = B * N
M = NS * P
EPS = 1e-5

_INTERP = False


def _pack_bf16(g):
    u = lax.bitcast_convert_type(g, jnp.uint32)
    r = (u + 0x7FFF + ((u >> 16) & 1)) >> 16
    lo = r[:, :C2]
    hi = r[:, C2:]
    return lax.bitcast_convert_type(lo | (hi << 16), jnp.float32)


def _unpack_bf16(gp):
    u = lax.bitcast_convert_type(gp, jnp.uint32)
    lo = lax.bitcast_convert_type(u << 16, jnp.float32)
    hi = lax.bitcast_convert_type(u & jnp.uint32(0xFFFF0000), jnp.float32)
    return jnp.concatenate([lo, hi], axis=-1)


def _proj_body(pos1_ref, pos2_ref, f1_ref, f2_ref, wp_ref, wf1_ref, wf2_ref,
               g_ref, h_ref):
    dn = (((0,), (1,)), ((), ()))
    g = lax.dot_general(f2_ref[0], wf2_ref[...], dn,
                        preferred_element_type=jnp.float32)
    g += lax.dot_general(pos2_ref[0], wp_ref[...], dn,
                         preferred_element_type=jnp.float32)
    g_ref[...] = g
    h = lax.dot_general(f1_ref[0], wf1_ref[...], dn,
                        preferred_element_type=jnp.float32)
    h -= lax.dot_general(pos1_ref[0], wp_ref[...], dn,
                         preferred_element_type=jnp.float32)
    h_ref[...] = h


def _proj(pos1, pos2, f1, f2, wp, wf1, wf2):
    return pl.pallas_call(
        _proj_body,
        grid=(B,),
        in_specs=[
            pl.BlockSpec((1, 3, N), lambda b: (b, 0, 0)),
            pl.BlockSpec((1, 3, N), lambda b: (b, 0, 0)),
            pl.BlockSpec((1, C, N), lambda b: (b, 0, 0)),
            pl.BlockSpec((1, C, N), lambda b: (b, 0, 0)),
            pl.BlockSpec((C, 3), lambda b: (0, 0)),
            pl.BlockSpec((C, C), lambda b: (0, 0)),
            pl.BlockSpec((C, C), lambda b: (0, 0)),
        ],
        out_specs=[
            pl.BlockSpec((N, C), lambda b: (b, 0)),
            pl.BlockSpec((N, C), lambda b: (b, 0)),
        ],
        out_shape=[
            jax.ShapeDtypeStruct((P, C), jnp.float32),
            jax.ShapeDtypeStruct((P, C), jnp.float32),
        ],
        interpret=_INTERP,
    )(pos1, pos2, f1, f2, wp, wf1, wf2)


_RB = 1024


def _knn_body(p1_ref, p2_ref, out_ref):
    b = pl.program_id(0)
    p1 = p1_ref[0]
    p2 = p2_ref[0]
    d = -2.0 * lax.dot_general(p1, p2, (((0,), (0,)), ((), ())),
                               preferred_element_type=jnp.float32)
    d += jnp.sum(p2 * p2, axis=0)[None, :]
    d = d.T
    iota = lax.broadcasted_iota(jnp.int32, (N, _RB), 0)
    inf = jnp.float32(jnp.inf)
    for s in range(NS):
        am = jnp.argmin(d, axis=0).astype(jnp.int32)
        out_ref[s, :] = am + b * N
        d = jnp.where(iota == am[None, :], inf, d)


def _knn(pos1, pos2):
    return pl.pallas_call(
        _knn_body,
        grid=(B, N // _RB),
        in_specs=[
            pl.BlockSpec((1, 3, _RB), lambda b, i: (b, 0, i)),
            pl.BlockSpec((1, 3, N), lambda b, i: (b, 0, 0)),
        ],
        out_specs=pl.BlockSpec((NS, _RB), lambda b, i: (0, b * (N // _RB) + i)),
        out_shape=jax.ShapeDtypeStruct((NS, P), jnp.int32),
        interpret=_INTERP,
    )(pos1, pos2)


_NC_SC, _NSUB_SC = 2, 16
_NW = _NC_SC * _NSUB_SC
_ROWS_W = M // _NW
_CHUNK = 128
_NCHUNK = _ROWS_W // _CHUNK


def _gather_sc(table, idx2d):
    mesh = plsc.VectorSubcoreMesh(core_axis_name="c", subcore_axis_name="s")

    @functools.partial(
        pl.kernel, mesh=mesh,
        out_type=jax.ShapeDtypeStruct((M, C), jnp.float32),
        scratch_types=[
            pltpu.VMEM((_NCHUNK, _CHUNK), jnp.int32),
            pltpu.VMEM((_CHUNK, C), jnp.float32),
            pltpu.VMEM((_CHUNK, C), jnp.float32),
            pltpu.SemaphoreType.DMA,
            pltpu.SemaphoreType.DMA,
        ],
    )
    def k(table_hbm, idx_hbm, out_hbm, idx_v, buf0, buf1, sem0, sem1):
        wid = lax.axis_index("s") * _NC_SC + lax.axis_index("c")
        pltpu.sync_copy(idx_hbm.at[pl.ds(wid * _NCHUNK, _NCHUNK)], idx_v)
        out_base = wid * _ROWS_W

        def body(j2, _):
            j0 = j2 * 2
            cp0 = pltpu.async_copy(table_hbm.at[idx_v.at[j0]], buf0, sem0)
            cp1 = pltpu.async_copy(table_hbm.at[idx_v.at[j0 + 1]], buf1, sem1)
            cp0.wait()
            pltpu.sync_copy(buf0, out_hbm.at[pl.ds(out_base + j0 * _CHUNK,
                                                   _CHUNK)])
            cp1.wait()
            pltpu.sync_copy(buf1, out_hbm.at[pl.ds(out_base + (j0 + 1) * _CHUNK,
                                                   _CHUNK)])
            return 0

        lax.fori_loop(0, _NCHUNK // 2, body, 0)

    return k(table, idx2d)


def _affine(g, b, s, q):
    mean = s / jnp.float32(M)
    var = q / jnp.float32(M) - mean * mean
    a = g * lax.rsqrt(var + EPS)
    c = b - mean * a
    return a, c


_PB = 8192

_STATS_OUT_SPECS = [
    pl.BlockSpec((1, C), lambda i, s: (0, 0)),
    pl.BlockSpec((1, C), lambda i, s: (0, 0)),
]
_STATS_OUT_SHAPE = [
    jax.ShapeDtypeStruct((1, C), jnp.float32),
    jax.ShapeDtypeStruct((1, C), jnp.float32),
]
_GG_SPEC = pl.BlockSpec((1, _PB, C), lambda i, s: (s, i, 0))
_GGP_SPEC = pl.BlockSpec((1, _PB, C2), lambda i, s: (s, i, 0))
_HT_SPEC = pl.BlockSpec((_PB, C), lambda i, s: (i, 0))
_W_SPEC = pl.BlockSpec((C, C), lambda i, s: (0, 0))
_V_SPEC = pl.BlockSpec((1, C), lambda i, s: (0, 0))


def _acc_stats(sum_ref, sq_ref, y):
    @pl.when((pl.program_id(0) == 0) & (pl.program_id(1) == 0))
    def _():
        sum_ref[...] = jnp.zeros_like(sum_ref)
        sq_ref[...] = jnp.zeros_like(sq_ref)

    sum_ref[0, :] += jnp.sum(y, axis=0)
    sq_ref[0, :] += jnp.sum(y * y, axis=0)


def _bn_relu_mm(y, w, g, b, s, q):
    a, c = _affine(g, b, s, q)
    x = jnp.maximum(y * a[None, :] + c[None, :], 0.0)
    return lax.dot_general(x, w, (((1,), (1,)), ((), ())),
                           preferred_element_type=jnp.float32)


def _stats1_body(gg_ref, ht_ref, sum_ref, sq_ref):
    _acc_stats(sum_ref, sq_ref, gg_ref[0] + ht_ref[...])


def _stats1(gg, ht):
    return pl.pallas_call(
        _stats1_body,
        grid=(P // _PB, NS),
        in_specs=[_GG_SPEC, _HT_SPEC],
        out_specs=_STATS_OUT_SPECS,
        out_shape=_STATS_OUT_SHAPE,
        interpret=_INTERP,
    )(gg.reshape(NS, P, C), ht)


def _stats2_body(gg_ref, ht_ref, w1_ref, g0_ref, b0_ref, s1_ref, q1_ref,
                 sum_ref, sq_ref):
    y2 = _bn_relu_mm(gg_ref[0] + ht_ref[...], w1_ref[...], g0_ref[0, :],
                     b0_ref[0, :], s1_ref[0, :], q1_ref[0, :])
    _acc_stats(sum_ref, sq_ref, y2)


def _stats2(gg, ht, w1, g0, b0, s1, q1):
    return pl.pallas_call(
        _stats2_body,
        grid=(P // _PB, NS),
        in_specs=[_GG_SPEC, _HT_SPEC, _W_SPEC] + [_V_SPEC] * 4,
        out_specs=_STATS_OUT_SPECS,
        out_shape=_STATS_OUT_SHAPE,
        interpret=_INTERP,
    )(gg.reshape(NS, P, C), ht, w1, g0, b0, s1, q1)


def _stats3_body(gg_ref, ht_ref, w1_ref, w2_ref, g0_ref, b0_ref, s1_ref,
                 q1_ref, g1_ref, b1_ref, s2_ref, q2_ref, sum_ref, sq_ref):
    y2 = _bn_relu_mm(gg_ref[0] + ht_ref[...], w1_ref[...], g0_ref[0, :],
                     b0_ref[0, :], s1_ref[0, :], q1_ref[0, :])
    y3 = _bn_relu_mm(y2, w2_ref[...], g1_ref[0, :], b1_ref[0, :],
                     s2_ref[0, :], q2_ref[0, :])
    _acc_stats(sum_ref, sq_ref, y3)


def _stats3(gg, ht, w1, w2, g0, b0, s1, q1, g1, b1, s2, q2):
    return pl.pallas_call(
        _stats3_body,
        grid=(P // _PB, NS),
        in_specs=[_GG_SPEC, _HT_SPEC, _W_SPEC, _W_SPEC] + [_V_SPEC] * 8,
        out_specs=_STATS_OUT_SPECS,
        out_shape=_STATS_OUT_SHAPE,
        interpret=_INTERP,
    )(gg.reshape(NS, P, C), ht, w1, w2, g0, b0, s1, q1, g1, b1, s2, q2)


_PB2 = 2048


def _final_body(gg_ref, ht_ref, w1_ref, w2_ref, g0_ref, b0_ref, s1_ref,
                q1_ref, g1_ref, b1_ref, s2_ref, q2_ref, g2_ref, b2_ref,
                s3_ref, q3_ref, out_ref):
    y1 = (gg_ref[...] + ht_ref[...][None, :, :]).reshape(NS * _PB2, C)
    y2 = _bn_relu_mm(y1, w1_ref[...], g0_ref[0, :], b0_ref[0, :],
                     s1_ref[0, :], q1_ref[0, :])
    y3 = _bn_relu_mm(y2, w2_ref[...], g1_ref[0, :], b1_ref[0, :],
                     s2_ref[0, :], q2_ref[0, :])
    a, c = _affine(g2_ref[0, :], b2_ref[0, :], s3_ref[0, :], q3_ref[0, :])
    x = jnp.maximum(y3 * a[None, :] + c[None, :], 0.0)
    r = jnp.max(x.reshape(NS, _PB2, C), axis=0)
    out_ref[0] = r.T


def _final(gg, ht, w1, w2, g0, b0, s1, q1, g1, b1, s2, q2, g2, b2, s3, q3):
    nb = N // _PB2
    v = pl.BlockSpec((1, C), lambda t: (0, 0))
    w = pl.BlockSpec((C, C), lambda t: (0, 0))
    return pl.pallas_call(
        _final_body,
        grid=(P // _PB2,),
        in_specs=[
            pl.BlockSpec((NS, _PB2, C), lambda t: (0, t, 0)),
            pl.BlockSpec((_PB2, C), lambda t: (t, 0)),
            w, w, v, v, v, v, v, v, v, v, v, v, v, v,
        ],
        out_specs=pl.BlockSpec((1, C, _PB2), lambda t: (t // nb, 0, t % nb)),
        out_shape=jax.ShapeDtypeStruct((B, C, N), jnp.float32),
        interpret=_INTERP,
    )(gg.reshape(NS, P, C), ht, w1, w2, g0, b0, s1, q1, g1, b1, s2, q2,
      g2, b2, s3, q3)


def kernel(pos1, pos2, feature1, feature2, W0, W1, W2, g0, b0, g1, b1, g2, b2):
    wp = W0[:, :3]
    wf2 = W0[:, 3:3 + C]
    wf1 = W0[:, 3 + C:]
    r = lambda v: v.reshape(1, C)
    g0r, b0r, g1r, b1r, g2r, b2r = r(g0), r(b0), r(g1), r(b1), r(g2), r(b2)

    gt, ht = _proj(pos1, pos2, feature1, feature2, wp, wf1, wf2)
    idxf = _knn(pos1, pos2)
    gg = _gather_sc(gt, idxf.reshape(M // _CHUNK, _CHUNK))
    s1, q1 = _stats1(gg, ht)
    s2, q2 = _stats2(gg, ht, W1, g0r, b0r, s1, q1)
    s3, q3 = _stats3(gg, ht, W1, W2, g0r, b0r, s1, q1, g1r, b1r, s2, q2)
    feat = _final(gg, ht, W1, W2, g0r, b0r, s1, q1, g1r, b1r, s2, q2,
                  g2r, b2r, s3, q3)
    return (pos1, feat)
